# Initial kernel scaffold; baseline (speedup 1.0000x reference)
import jax
import jax.numpy as jnp
from jax.experimental import pallas as pl


def kernel(x, edge_index, batch, W1, Wp, bp):
    raise NotImplementedError("write your pallas kernel here")


# speedup vs baseline: 117.3527x; 117.3527x over previous
import jax
import jax.numpy as jnp
from jax import lax
from jax.experimental import pallas as pl
from jax.experimental.pallas import tpu as pltpu
from jax.experimental.pallas import tpu_sc as plsc

_N = 10000
_E = 320000
_G = 64
_NC = 2
_NS = 16
_NW = _NC * _NS
_L = 16

_NPAD = 10240
_PT = _NPAD // _NS
_SL = _NPAD // _NW
_EPT = _E // _NW
_HBLK = 79

_mesh = plsc.VectorSubcoreMesh(
    core_axis_name="c", subcore_axis_name="s", num_cores=_NC, num_subcores=_NS
)

_sc_params = pltpu.CompilerParams(needs_layout_passes=False)


def _rsqrt16(x):
    i = plsc.bitcast(x, jnp.int32)
    y = plsc.bitcast(jnp.int32(0x5F3759DF) - (i >> 1), jnp.float32)
    for _ in range(3):
        y = y * (1.5 - 0.5 * x * y * y)
    return y


def _h_body(x_ref, w_ref, o_ref):
    o_ref[...] = lax.dot_general(
        x_ref[...], w_ref[...],
        dimension_numbers=(((1,), (1,)), ((), ())),
        preferred_element_type=jnp.float32,
    )


def _h_matvec(x, W1):
    return pl.pallas_call(
        _h_body,
        out_shape=jax.ShapeDtypeStruct((_N, 1), jnp.float32),
    )(x, W1)


def _deg_body(col_hbm, deg0_hbm, deg1_hbm, idx_buf, ones_v, zer_v, hist_sh):
    c = lax.axis_index("c")
    s = lax.axis_index("s")
    wid = c * _NS + s

    one16 = jnp.full((_L,), 1.0, jnp.float32)

    @pl.loop(0, 128 // _L)
    def _(k):
        ones_v[pl.ds(k * _L, _L)] = one16

    z16 = jnp.zeros((_L,), jnp.float32)

    @pl.loop(0, _PT // _L)
    def _(k):
        zer_v[pl.ds(k * _L, _L)] = z16

    pltpu.sync_copy(zer_v, hist_sh.at[pl.ds(pl.multiple_of(s * _PT, _PT), _PT)])

    pltpu.sync_copy(col_hbm.at[wid], idx_buf)

    plsc.subcore_barrier()

    @pl.loop(0, _HBLK)
    def _(j):
        pltpu.sync_copy(ones_v, hist_sh.at[idx_buf.at[j]], add=True)

    plsc.subcore_barrier()

    @pl.when(jnp.logical_and(s == 0, c == 0))
    def _():
        pltpu.sync_copy(hist_sh, deg0_hbm)

    @pl.when(jnp.logical_and(s == 0, c == 1))
    def _():
        pltpu.sync_copy(hist_sh, deg1_hbm)


def _deg_call(col3d):
    kern = pl.kernel(
        _deg_body,
        out_type=(
            jax.ShapeDtypeStruct((_NPAD,), jnp.float32),
            jax.ShapeDtypeStruct((_NPAD,), jnp.float32),
        ),
        mesh=_mesh,
        scratch_types=[
            pltpu.VMEM((_HBLK, 128), jnp.int32),
            pltpu.VMEM((128,), jnp.float32),
            pltpu.VMEM((_PT,), jnp.float32),
            pltpu.VMEM_SHARED((_NPAD,), jnp.float32),
        ],
        compiler_params=_sc_params,
    )
    return kern(col3d)


def _main_body(row_hbm, col_hbm, h_hbm, deg0_hbm, deg1_hbm, batch_hbm, out_hbm,
               row_v, col_v, bt_v, s_tab, d_tab,
               h_sl, d0, d1, s_sl, di_sl, s_sh, d_sh, bins,
               sem_r, sem_c, sem_b):
    c = lax.axis_index("c")
    s = lax.axis_index("s")
    wid = c * _NS + s

    eb = pl.multiple_of(wid * _EPT, _EPT)
    cp_r = pltpu.async_copy(row_hbm.at[pl.ds(eb, _EPT)], row_v, sem_r)
    cp_c = pltpu.async_copy(col_hbm.at[pl.ds(eb, _EPT)], col_v, sem_c)
    cp_b = pltpu.async_copy(batch_hbm, bt_v, sem_b)

    nb = pl.multiple_of(s * _PT, _PT)
    pltpu.sync_copy(h_hbm.at[pl.ds(nb, _PT)], h_sl)
    pltpu.sync_copy(deg0_hbm.at[pl.ds(nb, _PT)], d0)
    pltpu.sync_copy(deg1_hbm.at[pl.ds(nb, _PT)], d1)

    @pl.loop(0, _PT // _L)
    def _(k):
        sl = pl.ds(k * _L, _L)
        deg = d0[sl] + d1[sl] + 1.0
        y = _rsqrt16(deg)
        di_sl[sl] = y
        s_sl[sl] = h_sl[sl] * y

    pltpu.sync_copy(di_sl, d_sh.at[pl.ds(nb, _PT)])
    pltpu.sync_copy(s_sl, s_sh.at[pl.ds(nb, _PT)])
    plsc.subcore_barrier()

    pltpu.sync_copy(s_sh, s_tab)
    pltpu.sync_copy(d_sh, d_tab)

    z16 = jnp.zeros((_L,), jnp.float32)

    @pl.loop(0, _G)
    def _(g):
        bins[pl.ds(g * _L, _L)] = z16

    cp_r.wait()
    cp_c.wait()
    cp_b.wait()

    lane = lax.iota(jnp.int32, _L)

    @pl.loop(0, _EPT // _L)
    def _(j):
        sl = pl.ds(j * _L, _L)
        r = row_v[sl]
        cc = col_v[sl]
        sv = plsc.load_gather(s_tab, [r])
        dv = plsc.load_gather(d_tab, [cc])
        bv = plsc.load_gather(bt_v, [cc])
        plsc.addupdate_scatter(bins, [(bv << 4) + lane], sv * dv)

    sb = pl.multiple_of(wid * _SL, _SL)

    @pl.loop(0, _SL // _L)
    def _(k):
        sl = pl.ds(sb + k * _L, _L)
        val = s_tab[sl] * d_tab[sl]
        plsc.addupdate_scatter(bins, [(bt_v[sl] << 4) + lane], val)

    pltpu.sync_copy(
        bins, out_hbm.at[pl.ds(pl.multiple_of(wid * _G * _L, _G * _L), _G * _L)])


def _main_call(row1d, col1d, h_pad, deg0, deg1, batch_pad):
    kern = pl.kernel(
        _main_body,
        out_type=jax.ShapeDtypeStruct((_NW * _G * _L,), jnp.float32),
        mesh=_mesh,
        scratch_types=[
            pltpu.VMEM((_EPT,), jnp.int32),
            pltpu.VMEM((_EPT,), jnp.int32),
            pltpu.VMEM((_NPAD,), jnp.int32),
            pltpu.VMEM((_NPAD,), jnp.float32),
            pltpu.VMEM((_NPAD,), jnp.float32),
            pltpu.VMEM((_PT,), jnp.float32),
            pltpu.VMEM((_PT,), jnp.float32),
            pltpu.VMEM((_PT,), jnp.float32),
            pltpu.VMEM((_PT,), jnp.float32),
            pltpu.VMEM((_PT,), jnp.float32),
            pltpu.VMEM_SHARED((_NPAD,), jnp.float32),
            pltpu.VMEM_SHARED((_NPAD,), jnp.float32),
            pltpu.VMEM((_G * _L,), jnp.float32),
            pltpu.SemaphoreType.DMA,
            pltpu.SemaphoreType.DMA,
            pltpu.SemaphoreType.DMA,
        ],
        compiler_params=_sc_params,
    )
    return kern(row1d, col1d, h_pad, deg0, deg1, batch_pad)


def _final_body(b_ref, wp_ref, bp_ref, o_ref):
    t = jnp.sum(b_ref[...], axis=(0, 2))[:, None]
    o_ref[...] = t * wp_ref[...] + bp_ref[...][None, :]


def _final_call(bins3d, Wp, bp):
    return pl.pallas_call(
        _final_body,
        out_shape=jax.ShapeDtypeStruct((_G, 1), jnp.float32),
    )(bins3d, Wp, bp)


def kernel(x, edge_index, batch, W1, Wp, bp):
    h = _h_matvec(x, W1)
    h_pad = jnp.pad(h[:, 0], (0, _NPAD - _N))
    col1d = edge_index[1]
    col3d = jnp.pad(col1d, (0, _NW * _HBLK * 128 - _E),
                    constant_values=_NPAD - 1).reshape(_NW, _HBLK, 128)
    deg0, deg1 = _deg_call(col3d)
    batch_pad = jnp.pad(batch, (0, _NPAD - _N))
    bins = _main_call(edge_index[0], col1d, h_pad, deg0, deg1, batch_pad)
    return _final_call(bins.reshape(_NW, _G, _L), Wp, bp)
